# Initial kernel scaffold; baseline (speedup 1.0000x reference)
import jax
import jax.numpy as jnp
from jax.experimental import pallas as pl


def kernel(modality_tokens, timestamps, channel_embed, pos_embed, month_tab, patch_size, input_res):
    raise NotImplementedError("write your pallas kernel here")


# speedup vs baseline: 2.7085x; 2.7085x over previous
import functools

import jax
import jax.numpy as jnp
from jax import lax
from jax.experimental import pallas as pl
from jax.experimental.pallas import tpu as pltpu
from jax.experimental.pallas import tpu_sc as plsc

B, H, W, T, BS, D = 8, 14, 14, 12, 3, 768
N = D // 4
HW = H * W
SITES = B * HW
ROWS = T * BS
U_W = 3 * N
LN10K = 9.210340371976184


def _tables_body(gsd_ref, months_ref, ch_ref, pos_ref, mt_ref, u_ref, se_ref):
    months = months_ref[...]
    mk3 = lax.broadcast_in_dim(months, (B, T, N), (0, 1))
    memb = jnp.zeros((B, T, N), jnp.float32)
    for k in range(13):
        row = lax.broadcast_in_dim(mt_ref[k, :], (B, T, N), (2,))
        memb = memb + jnp.where(mk3 == k, row, 0.0)
    chb = lax.broadcast_in_dim(ch_ref[...], (B, T, BS, N), (2, 3))
    posb = lax.broadcast_in_dim(pos_ref[...][:T], (B, T, BS, N), (1, 3))
    membb = lax.broadcast_in_dim(memb, (B, T, BS, N), (0, 1, 3))
    u_ref[...] = jnp.concatenate([chb, posb, membb], axis=-1)

    gsd = gsd_ref[0, 0]
    ri = lax.broadcasted_iota(jnp.int32, (HW, N // 4), 0)
    ki = lax.broadcasted_iota(jnp.int32, (HW, N // 4), 1).astype(jnp.float32)
    omega = jnp.exp(ki * (-LN10K / (N // 4)))
    py = (ri // W).astype(jnp.float32) * gsd
    px = (ri % W).astype(jnp.float32) * gsd
    oy = py * omega
    ox = px * omega
    se_ref[...] = jnp.concatenate(
        [jnp.sin(oy), jnp.cos(oy), jnp.sin(ox), jnp.cos(ox)], axis=-1)


def _build_tables(gsd, months, channel_embed, pos_embed, month_tab):
    return pl.pallas_call(
        _tables_body,
        out_shape=(
            jax.ShapeDtypeStruct((B, T, BS, U_W), jnp.float32),
            jax.ShapeDtypeStruct((HW, N), jnp.float32),
        ),
        in_specs=[
            pl.BlockSpec(memory_space=pltpu.SMEM),
            pl.BlockSpec(memory_space=pltpu.VMEM),
            pl.BlockSpec(memory_space=pltpu.VMEM),
            pl.BlockSpec(memory_space=pltpu.VMEM),
            pl.BlockSpec(memory_space=pltpu.VMEM),
        ],
    )(gsd, months, channel_embed, pos_embed, month_tab)


SITES_PER_WORKER = SITES // 32


def _sc_add_body(tok_hbm, u_hbm, se_hbm, out_hbm, u_v, se_v, tok_v):
    c = lax.axis_index("c")
    s = lax.axis_index("s")
    wid = c * 16 + s
    bidx = wid // 4
    q = wid % 4
    pltpu.sync_copy(u_hbm.at[bidx], u_v)
    pltpu.sync_copy(se_hbm.at[q], se_v)

    def unit_body(k, carry):
        site = bidx * HW + q * SITES_PER_WORKER + k
        pltpu.sync_copy(tok_hbm.at[site], tok_v)
        sev = [se_v[k, pl.ds(i * 16, 16)] for i in range(N // 16)]

        def row_body(j, c2):
            for i in range(U_W // 16):
                plsc.addupdate(tok_v.at[j, pl.ds(i * 16, 16)],
                               u_v[j, pl.ds(i * 16, 16)])
            for i in range(N // 16):
                plsc.addupdate(tok_v.at[j, pl.ds(U_W + i * 16, 16)], sev[i])
            return c2

        lax.fori_loop(0, ROWS, row_body, 0)
        pltpu.sync_copy(tok_v, out_hbm.at[site])
        return carry

    lax.fori_loop(0, SITES_PER_WORKER, unit_body, 0)


@functools.cache
def _sc_add():
    return functools.partial(
        pl.kernel,
        out_type=jax.ShapeDtypeStruct((SITES, ROWS, D), jnp.float32),
        mesh=plsc.VectorSubcoreMesh(core_axis_name="c", subcore_axis_name="s",
                                    num_cores=2, num_subcores=16),
        scratch_types=[
            pltpu.VMEM((ROWS, U_W), jnp.float32),
            pltpu.VMEM((SITES_PER_WORKER, N), jnp.float32),
            pltpu.VMEM((ROWS, D), jnp.float32),
        ],
    )(_sc_add_body)


def kernel(modality_tokens, timestamps, channel_embed, pos_embed, month_tab,
           patch_size, input_res):
    gsd = (jnp.float32(input_res) * jnp.float32(patch_size) / 10.0).reshape(1, 1)
    months = timestamps[:, :, 1].astype(jnp.int32)
    u, se = _build_tables(gsd, months, channel_embed, pos_embed, month_tab)
    tok3 = modality_tokens.reshape(SITES, ROWS, D)
    u3 = u.reshape(B, ROWS, U_W)
    se4 = se.reshape(4, SITES_PER_WORKER, N)
    out3 = _sc_add()(tok3, u3, se4)
    return out3.reshape(modality_tokens.shape)


# speedup vs baseline: 4.6776x; 1.7270x over previous
import functools

import jax
import jax.numpy as jnp
from jax import lax
from jax.experimental import pallas as pl
from jax.experimental.pallas import tpu as pltpu
from jax.experimental.pallas import tpu_sc as plsc

B, H, W, T, BS, D = 8, 14, 14, 12, 3, 768
N = D // 4
HW = H * W
SITES = B * HW
U_W = 3 * N
NWORKERS = 32
SPW = HW // 4
LN10K = 9.210340371976184


def _tables_body(gsd_ref, months_ref, ch_ref, pos_ref, mt_ref, u_ref, se_ref):
    months = months_ref[...]
    mk3 = lax.broadcast_in_dim(months, (B, T, N), (0, 1))
    memb = jnp.zeros((B, T, N), jnp.float32)
    for k in range(13):
        row = lax.broadcast_in_dim(mt_ref[k, :], (B, T, N), (2,))
        memb = memb + jnp.where(mk3 == k, row, 0.0)
    chb = lax.broadcast_in_dim(ch_ref[...], (B, T, BS, N), (2, 3))
    posb = lax.broadcast_in_dim(pos_ref[...][:T], (B, T, BS, N), (1, 3))
    membb = lax.broadcast_in_dim(memb, (B, T, BS, N), (0, 1, 3))
    u_ref[...] = jnp.concatenate([chb, posb, membb], axis=-1)

    gsd = gsd_ref[0, 0]
    qq = lax.broadcasted_iota(jnp.int32, (4, SPW, N // 4), 0)
    kk = lax.broadcasted_iota(jnp.int32, (4, SPW, N // 4), 1)
    site = qq * SPW + kk
    ki = lax.broadcasted_iota(jnp.int32, (4, SPW, N // 4), 2).astype(jnp.float32)
    omega = jnp.exp(ki * (-LN10K / (N // 4)))
    py = (site // W).astype(jnp.float32) * gsd
    px = (site % W).astype(jnp.float32) * gsd
    oy = py * omega
    ox = px * omega
    se_ref[...] = jnp.concatenate(
        [jnp.sin(oy), jnp.cos(oy), jnp.sin(ox), jnp.cos(ox)], axis=-1)


def _build_tables(gsd, months, channel_embed, pos_embed, month_tab):
    return pl.pallas_call(
        _tables_body,
        out_shape=(
            jax.ShapeDtypeStruct((B, T, BS, U_W), jnp.float32),
            jax.ShapeDtypeStruct((4, SPW, N), jnp.float32),
        ),
        in_specs=[
            pl.BlockSpec(memory_space=pltpu.SMEM),
            pl.BlockSpec(memory_space=pltpu.VMEM),
            pl.BlockSpec(memory_space=pltpu.VMEM),
            pl.BlockSpec(memory_space=pltpu.VMEM),
            pl.BlockSpec(memory_space=pltpu.VMEM),
        ],
    )(gsd, months, channel_embed, pos_embed, month_tab)


def _sc_add_body(tok_hbm, u_hbm, se_hbm, out_hbm, u_v, se_v, tok_v):
    c = lax.axis_index("c")
    s = lax.axis_index("s")
    wid = c * 16 + s
    bidx = wid // 4
    q = wid % 4
    pltpu.sync_copy(u_hbm.at[bidx], u_v)
    pltpu.sync_copy(se_hbm.at[q], se_v)

    def unit_body(k, carry):
        hwsite = q * SPW + k
        hh = hwsite // W
        ww = hwsite % W
        pltpu.sync_copy(tok_hbm.at[bidx, hh, ww], tok_v)
        sev = [se_v[k, pl.ds(i * 16, 16)] for i in range(N // 16)]

        def row_body(j, c2):
            for bsi in range(BS):
                for i in range(U_W // 16):
                    plsc.addupdate(tok_v.at[j, bsi, pl.ds(i * 16, 16)],
                                   u_v[j, bsi, pl.ds(i * 16, 16)])
                for i in range(N // 16):
                    plsc.addupdate(tok_v.at[j, bsi, pl.ds(U_W + i * 16, 16)],
                                   sev[i])
            return c2

        lax.fori_loop(0, T, row_body, 0)
        pltpu.sync_copy(tok_v, out_hbm.at[bidx, hh, ww])
        return carry

    lax.fori_loop(0, SPW, unit_body, 0)


@functools.cache
def _sc_add():
    return functools.partial(
        pl.kernel,
        out_type=jax.ShapeDtypeStruct((B, H, W, T, BS, D), jnp.float32),
        mesh=plsc.VectorSubcoreMesh(core_axis_name="c", subcore_axis_name="s",
                                    num_cores=2, num_subcores=16),
        scratch_types=[
            pltpu.VMEM((T, BS, U_W), jnp.float32),
            pltpu.VMEM((SPW, N), jnp.float32),
            pltpu.VMEM((T, BS, D), jnp.float32),
        ],
    )(_sc_add_body)


def kernel(modality_tokens, timestamps, channel_embed, pos_embed, month_tab,
           patch_size, input_res):
    gsd = (jnp.float32(input_res) * jnp.float32(patch_size) / 10.0).reshape(1, 1)
    months = timestamps[:, :, 1].astype(jnp.int32)
    u, se = _build_tables(gsd, months, channel_embed, pos_embed, month_tab)
    return _sc_add()(modality_tokens, u, se)
